# baseline (speedup 1.0000x reference)
import jax
import jax.numpy as jnp
from jax import lax
from jax.experimental import pallas as pl
from jax.experimental.pallas import tpu as pltpu
from jax.experimental.pallas import tpu_sc as plsc

NC, NS = 2, 16
NW = NC * NS
BATCH = 16384
HIST = 50
EMBED = 32
VOCAB = 1000000
SRW = 128
NSR = VOCAB * EMBED // SRW
NB = 128
NBB = BATCH // NB
BPW = NBB // NW
RH = 10
NHG = HIST // RH


def _body(tbl_hbm, xT_hbm, out_hbm, idx_v, sr_v, gb_v, slab_v, gsem, osem):
    wid = lax.axis_index("s") * NC + lax.axis_index("c")
    lane = lax.iota(jnp.int32, 16)

    def fire(h, buf):
        pltpu.async_copy(tbl_hbm.at[sr_v.at[h]], gb_v.at[buf], gsem.at[buf])

    def wait(h, buf):
        pltpu.make_async_copy(
            tbl_hbm.at[sr_v.at[h]], gb_v.at[buf], gsem.at[buf]
        ).wait()

    def extract(i, h, buf):
        for j16 in range(NB // 16):
            dr16 = idx_v[h, pl.ds(j16 * 16, 16)]
            jb = j16 * 16 + lane
            for e in range(EMBED):
                vals = plsc.load_gather(gb_v.at[buf], [jb, dr16 + e])
                slab_v[i, e, pl.ds(j16 * 16, 16)] = vals

    def do_block(k, _):
        b0 = (wid * BPW + k) * NB
        pltpu.sync_copy(xT_hbm.at[:, pl.ds(b0, NB)], idx_v)

        def prep(i, _):
            row = i // 8
            c = (i % 8) * 16
            v = idx_v[row, pl.ds(c, 16)]
            sr_v[row, pl.ds(c, 16)] = lax.shift_right_logical(v, 2)
            idx_v[row, pl.ds(c, 16)] = lax.shift_left(
                lax.bitwise_and(v, 3), 5
            )
            return 0

        lax.fori_loop(0, HIST * (NB // 16), prep, 0)

        fire(0, 0)

        def do_slab(hg, _):
            h0 = hg * RH

            def pair(i2, _):
                h = h0 + i2 * 2
                wait(h, 0)

                @pl.when(h + 1 < HIST)
                def _():
                    fire(h + 1, 1)

                extract(i2 * 2, h, 0)
                wait(h + 1, 1)

                @pl.when(h + 2 < HIST)
                def _():
                    fire(h + 2, 0)

                extract(i2 * 2 + 1, h + 1, 1)
                return 0

            lax.fori_loop(0, RH // 2, pair, 0)
            pltpu.async_copy(
                slab_v, out_hbm.at[pl.ds(h0, RH), :, pl.ds(b0, NB)], osem
            )
            return 0

        def do_slab_waited(hg, _):
            @pl.when(hg + k * NHG != 0)
            def _():
                pltpu.make_async_copy(
                    slab_v, out_hbm.at[pl.ds(0, RH), :, pl.ds(0, NB)], osem
                ).wait()

            return do_slab(hg, 0)

        lax.fori_loop(0, NHG, do_slab_waited, 0)
        return 0

    lax.fori_loop(0, BPW, do_block, 0)
    pltpu.make_async_copy(
        slab_v, out_hbm.at[pl.ds(0, RH), :, pl.ds(0, NB)], osem
    ).wait()


@jax.jit
def _gather(xT, tblS):
    mesh = plsc.VectorSubcoreMesh(
        core_axis_name="c", subcore_axis_name="s",
        num_cores=NC, num_subcores=NS,
    )
    return pl.kernel(
        _body,
        out_type=jax.ShapeDtypeStruct((HIST, EMBED, BATCH), jnp.float32),
        mesh=mesh,
        scratch_types=[
            pltpu.VMEM((HIST, NB), jnp.int32),
            pltpu.VMEM((HIST, NB), jnp.int32),
            pltpu.VMEM((2, NB, SRW), jnp.float32),
            pltpu.VMEM((RH, EMBED, NB), jnp.float32),
            pltpu.SemaphoreType.DMA((2,)),
            pltpu.SemaphoreType.DMA,
        ],
        compiler_params=pltpu.CompilerParams(
            use_tc_tiling_on_sc=True, needs_layout_passes=False
        ),
    )(tblS, xT)


def kernel(x, table):
    tblS = table.reshape(NSR, SRW)
    xT = x.T
    outT = _gather(xT, tblS)
    return jnp.transpose(outT, (2, 0, 1))


# speedup vs baseline: 1.0346x; 1.0346x over previous
import jax
import jax.numpy as jnp
from jax import lax
from jax.experimental import pallas as pl
from jax.experimental.pallas import tpu as pltpu
from jax.experimental.pallas import tpu_sc as plsc

NC, NS = 2, 16
NW = NC * NS
BATCH = 16384
HIST = 50
EMBED = 32
VOCAB = 1000000
SRW = 128
NSR = VOCAB * EMBED // SRW
NB = 128
NBB = BATCH // NB
BPW = NBB // NW
RH = 10
NHG = HIST // RH


def _body(tbl_hbm, xT_hbm, out_hbm, idx_v, sr_v, gb_v, slab_v, gsem, osem):
    wid = lax.axis_index("s") * NC + lax.axis_index("c")
    lane = lax.iota(jnp.int32, 16)

    def fire(h, buf):
        pltpu.async_copy(tbl_hbm.at[sr_v.at[h]], gb_v.at[buf], gsem.at[buf])

    def wait(h, buf):
        pltpu.make_async_copy(
            tbl_hbm.at[sr_v.at[h]], gb_v.at[buf], gsem.at[buf]
        ).wait()

    def extract(i, h, buf):
        def chunk(j16, _):
            dr16 = idx_v[h, pl.ds(j16 * 16, 16)]
            jb = j16 * 16 + lane
            for e in range(EMBED):
                vals = plsc.load_gather(gb_v.at[buf], [jb, dr16 + e])
                slab_v[i, e, pl.ds(j16 * 16, 16)] = vals
            return 0

        lax.fori_loop(0, NB // 16, chunk, 0)

    def do_block(k, _):
        b0 = (wid * BPW + k) * NB
        pltpu.sync_copy(xT_hbm.at[:, pl.ds(b0, NB)], idx_v)

        def prep(i, _):
            row = i // 8
            c = (i % 8) * 16
            v = idx_v[row, pl.ds(c, 16)]
            sr_v[row, pl.ds(c, 16)] = lax.shift_right_logical(v, 2)
            idx_v[row, pl.ds(c, 16)] = lax.shift_left(
                lax.bitwise_and(v, 3), 5
            )
            return 0

        lax.fori_loop(0, HIST * (NB // 16), prep, 0)

        fire(0, 0)

        def do_slab(hg, _):
            h0 = hg * RH

            def pair(i2, _):
                h = h0 + i2 * 2
                wait(h, 0)

                @pl.when(h + 1 < HIST)
                def _():
                    fire(h + 1, 1)

                extract(i2 * 2, h, 0)
                wait(h + 1, 1)

                @pl.when(h + 2 < HIST)
                def _():
                    fire(h + 2, 0)

                extract(i2 * 2 + 1, h + 1, 1)
                return 0

            lax.fori_loop(0, RH // 2, pair, 0)
            pltpu.async_copy(
                slab_v, out_hbm.at[pl.ds(h0, RH), :, pl.ds(b0, NB)], osem
            )
            return 0

        def do_slab_waited(hg, _):
            @pl.when(hg + k * NHG != 0)
            def _():
                pltpu.make_async_copy(
                    slab_v, out_hbm.at[pl.ds(0, RH), :, pl.ds(0, NB)], osem
                ).wait()

            return do_slab(hg, 0)

        lax.fori_loop(0, NHG, do_slab_waited, 0)
        return 0

    lax.fori_loop(0, BPW, do_block, 0)
    pltpu.make_async_copy(
        slab_v, out_hbm.at[pl.ds(0, RH), :, pl.ds(0, NB)], osem
    ).wait()


@jax.jit
def _gather(xT, tblS):
    mesh = plsc.VectorSubcoreMesh(
        core_axis_name="c", subcore_axis_name="s",
        num_cores=NC, num_subcores=NS,
    )
    return pl.kernel(
        _body,
        out_type=jax.ShapeDtypeStruct((HIST, EMBED, BATCH), jnp.float32),
        mesh=mesh,
        scratch_types=[
            pltpu.VMEM((HIST, NB), jnp.int32),
            pltpu.VMEM((HIST, NB), jnp.int32),
            pltpu.VMEM((2, NB, SRW), jnp.float32),
            pltpu.VMEM((RH, EMBED, NB), jnp.float32),
            pltpu.SemaphoreType.DMA((2,)),
            pltpu.SemaphoreType.DMA,
        ],
        compiler_params=pltpu.CompilerParams(
            use_tc_tiling_on_sc=True, needs_layout_passes=False
        ),
    )(tblS, xT)


def kernel(x, table):
    tblS = table.reshape(NSR, SRW)
    xT = x.T
    outT = _gather(xT, tblS)
    return jnp.transpose(outT, (2, 0, 1))


# speedup vs baseline: 1.4641x; 1.4152x over previous
import jax
import jax.numpy as jnp
from jax import lax
from jax.experimental import pallas as pl
from jax.experimental.pallas import tpu as pltpu
from jax.experimental.pallas import tpu_sc as plsc

NC, NS = 2, 16
NW = NC * NS
BATCH = 16384
HIST = 50
EMBED = 32
VOCAB = 1000000
SRW = 128
NSR = VOCAB * EMBED // SRW
NB = 128
NBB = BATCH // NB
BPW = NBB // NW
RH = 10
NHG = HIST // RH


def _body(tbl_hbm, xT_hbm, out_hbm, idx_v, sr_v, gb_v, slab_v, gsem, osem):
    wid = lax.axis_index("s") * NC + lax.axis_index("c")
    lane = lax.iota(jnp.int32, 16)

    def fire(h, buf):
        pltpu.async_copy(tbl_hbm.at[sr_v.at[h]], gb_v.at[buf], gsem.at[buf])

    def wait(h, buf):
        pltpu.make_async_copy(
            tbl_hbm.at[sr_v.at[h]], gb_v.at[buf], gsem.at[buf]
        ).wait()

    def extract(i, h, buf):
        def chunk(j16, _):
            dr16 = idx_v[h, pl.ds(j16 * 16, 16)]
            jb = j16 * 16 + lane
            for c in range(EMBED // 16):
                base = dr16 + c * 16
                for d in range(16):
                    ecol = lax.bitwise_and(lane + d, 15)
                    vals = plsc.load_gather(gb_v.at[buf], [jb, base + ecol])
                    plsc.store_scatter(
                        slab_v.at[i], [c * 16 + ecol, jb], vals
                    )
            return 0

        lax.fori_loop(0, NB // 16, chunk, 0)

    def do_block(k, _):
        b0 = (wid * BPW + k) * NB
        pltpu.sync_copy(xT_hbm.at[:, pl.ds(b0, NB)], idx_v)

        def prep(i, _):
            row = i // 8
            c = (i % 8) * 16
            v = idx_v[row, pl.ds(c, 16)]
            sr_v[row, pl.ds(c, 16)] = lax.shift_right_logical(v, 2)
            idx_v[row, pl.ds(c, 16)] = lax.shift_left(
                lax.bitwise_and(v, 3), 5
            )
            return 0

        lax.fori_loop(0, HIST * (NB // 16), prep, 0)

        fire(0, 0)

        def do_slab(hg, _):
            h0 = hg * RH

            def pair(i2, _):
                h = h0 + i2 * 2
                wait(h, 0)

                @pl.when(h + 1 < HIST)
                def _():
                    fire(h + 1, 1)

                extract(i2 * 2, h, 0)
                wait(h + 1, 1)

                @pl.when(h + 2 < HIST)
                def _():
                    fire(h + 2, 0)

                extract(i2 * 2 + 1, h + 1, 1)
                return 0

            lax.fori_loop(0, RH // 2, pair, 0)
            pltpu.async_copy(
                slab_v, out_hbm.at[pl.ds(h0, RH), :, pl.ds(b0, NB)], osem
            )
            return 0

        def do_slab_waited(hg, _):
            @pl.when(hg + k * NHG != 0)
            def _():
                pltpu.make_async_copy(
                    slab_v, out_hbm.at[pl.ds(0, RH), :, pl.ds(0, NB)], osem
                ).wait()

            return do_slab(hg, 0)

        lax.fori_loop(0, NHG, do_slab_waited, 0)
        return 0

    lax.fori_loop(0, BPW, do_block, 0)
    pltpu.make_async_copy(
        slab_v, out_hbm.at[pl.ds(0, RH), :, pl.ds(0, NB)], osem
    ).wait()


@jax.jit
def _gather(xT, tblS):
    mesh = plsc.VectorSubcoreMesh(
        core_axis_name="c", subcore_axis_name="s",
        num_cores=NC, num_subcores=NS,
    )
    return pl.kernel(
        _body,
        out_type=jax.ShapeDtypeStruct((HIST, EMBED, BATCH), jnp.float32),
        mesh=mesh,
        scratch_types=[
            pltpu.VMEM((HIST, NB), jnp.int32),
            pltpu.VMEM((HIST, NB), jnp.int32),
            pltpu.VMEM((2, NB, SRW), jnp.float32),
            pltpu.VMEM((RH, EMBED, NB), jnp.float32),
            pltpu.SemaphoreType.DMA((2,)),
            pltpu.SemaphoreType.DMA,
        ],
        compiler_params=pltpu.CompilerParams(
            use_tc_tiling_on_sc=True, needs_layout_passes=False
        ),
    )(tblS, xT)


def kernel(x, table):
    tblS = table.reshape(NSR, SRW)
    xT = x.T
    outT = _gather(xT, tblS)
    return jnp.transpose(outT, (2, 0, 1))


# speedup vs baseline: 1.4963x; 1.0220x over previous
import jax
import jax.numpy as jnp
from jax import lax
from jax.experimental import pallas as pl
from jax.experimental.pallas import tpu as pltpu
from jax.experimental.pallas import tpu_sc as plsc

NC, NS = 2, 16
NW = NC * NS
BATCH = 16384
HIST = 50
EMBED = 32
VOCAB = 1000000
SRW = 128
NSR = VOCAB * EMBED // SRW
NB = 128
NBB = BATCH // NB
BPW = NBB // NW
RH = 10
NHG = HIST // RH


def _body(tbl_hbm, xT_hbm, out_hbm, idx_v, sr_v, gb_v, slab_v, gsem, osem):
    wid = lax.axis_index("s") * NC + lax.axis_index("c")
    lane = lax.iota(jnp.int32, 16)

    def fire(h, buf):
        pltpu.async_copy(tbl_hbm.at[sr_v.at[h]], gb_v.at[buf], gsem.at[buf])

    def wait(h, buf):
        pltpu.make_async_copy(
            tbl_hbm.at[sr_v.at[h]], gb_v.at[buf], gsem.at[buf]
        ).wait()

    def extract(i, h, buf, sb):
        def chunk(j16, _):
            dr16 = idx_v[h, pl.ds(j16 * 16, 16)]
            jb = j16 * 16 + lane
            for c in range(EMBED // 16):
                base = dr16 + c * 16
                for d in range(16):
                    ecol = lax.bitwise_and(lane + d, 15)
                    vals = plsc.load_gather(gb_v.at[buf], [jb, base + ecol])
                    plsc.store_scatter(
                        slab_v.at[sb, i], [c * 16 + ecol, jb], vals
                    )
            return 0

        lax.fori_loop(0, NB // 16, chunk, 0)

    def do_block(k, _):
        b0 = (wid * BPW + k) * NB
        pltpu.sync_copy(xT_hbm.at[:, pl.ds(b0, NB)], idx_v)

        def prep(i, _):
            row = i // 8
            c = (i % 8) * 16
            v = idx_v[row, pl.ds(c, 16)]
            sr_v[row, pl.ds(c, 16)] = lax.shift_right_logical(v, 2)
            idx_v[row, pl.ds(c, 16)] = lax.shift_left(
                lax.bitwise_and(v, 3), 5
            )
            return 0

        lax.fori_loop(0, NB // 16, prep, 0)
        fire(0, 0)
        lax.fori_loop(NB // 16, HIST * (NB // 16), prep, 0)

        def do_slab(hg, _):
            g = k * NHG + hg
            sb = lax.bitwise_and(g, 1)
            h0 = hg * RH

            @pl.when(g >= 2)
            def _():
                pltpu.make_async_copy(
                    slab_v.at[sb],
                    out_hbm.at[pl.ds(0, RH), :, pl.ds(0, NB)],
                    osem.at[sb],
                ).wait()

            def pair(i2, _):
                h = h0 + i2 * 2
                wait(h, 0)

                @pl.when(h + 1 < HIST)
                def _():
                    fire(h + 1, 1)

                extract(i2 * 2, h, 0, sb)
                wait(h + 1, 1)

                @pl.when(h + 2 < HIST)
                def _():
                    fire(h + 2, 0)

                extract(i2 * 2 + 1, h + 1, 1, sb)
                return 0

            lax.fori_loop(0, RH // 2, pair, 0)
            pltpu.async_copy(
                slab_v.at[sb],
                out_hbm.at[pl.ds(h0, RH), :, pl.ds(b0, NB)],
                osem.at[sb],
            )
            return 0

        lax.fori_loop(0, NHG, do_slab, 0)
        return 0

    lax.fori_loop(0, BPW, do_block, 0)
    for s in range(2):
        pltpu.make_async_copy(
            slab_v.at[s], out_hbm.at[pl.ds(0, RH), :, pl.ds(0, NB)],
            osem.at[s],
        ).wait()


@jax.jit
def _gather(xT, tblS):
    mesh = plsc.VectorSubcoreMesh(
        core_axis_name="c", subcore_axis_name="s",
        num_cores=NC, num_subcores=NS,
    )
    return pl.kernel(
        _body,
        out_type=jax.ShapeDtypeStruct((HIST, EMBED, BATCH), jnp.float32),
        mesh=mesh,
        scratch_types=[
            pltpu.VMEM((HIST, NB), jnp.int32),
            pltpu.VMEM((HIST, NB), jnp.int32),
            pltpu.VMEM((2, NB, SRW), jnp.float32),
            pltpu.VMEM((2, RH, EMBED, NB), jnp.float32),
            pltpu.SemaphoreType.DMA((2,)),
            pltpu.SemaphoreType.DMA((2,)),
        ],
        compiler_params=pltpu.CompilerParams(
            use_tc_tiling_on_sc=True, needs_layout_passes=False
        ),
    )(tblS, xT)


def kernel(x, table):
    tblS = table.reshape(NSR, SRW)
    xT = x.T
    outT = _gather(xT, tblS)
    return jnp.transpose(outT, (2, 0, 1))
